# baseline (speedup 1.0000x reference)
import functools

import jax
import jax.numpy as jnp
from jax import lax
from jax.experimental import pallas as pl
from jax.experimental.pallas import tpu as pltpu
from jax.experimental.pallas import tpu_sc as plsc

B = 4
N = 2048
M = 2048
CIN = 256
D = 128
K = 16
DH = 512
TOT = B * N * K

_HI = lax.Precision.HIGHEST
_F32 = jnp.float32


def _dot(a, b, dims):
    return lax.dot_general(a, b, (dims, ((), ())), precision=_HI,
                           preferred_element_type=_F32)


def _dotbf(a, b, dims):
    return lax.dot_general(a.astype(jnp.bfloat16), b.astype(jnp.bfloat16),
                           (dims, ((), ())), preferred_element_type=_F32)


def _bf_bits(x):
    return lax.bitcast_convert_type(
        x.astype(jnp.bfloat16).astype(_F32), jnp.int32)


def _p1_body(fq, fs, pq, ps, qw, qb, kw, kb, vw, vb, e3,
             qt_ref, kvt_ref, pqt_ref, pst_ref):
    f_q = fq[0]
    f_s = fs[0]
    qt_ref[0] = _dotbf(f_q, qw[...], ((0,), (1,))) + qb[...]
    kres = _dotbf(f_s, kw[...], ((0,), (1,))) + kb[...]
    vres = _dotbf(f_s, vw[...], ((0,), (1,))) + vb[...]
    kvt_ref[0] = jnp.bitwise_or(
        _bf_bits(kres), lax.shift_right_logical(_bf_bits(vres), 16))
    pqt_ref[0] = _dot(pq[0], e3[...], ((0,), (0,)))
    pst_ref[0] = _dot(ps[0], e3[...], ((0,), (0,)))


def _p1(fq, fs, pq, ps, qw, qb, kw, kb, vw, vb):
    e3 = jnp.eye(3, 16, dtype=_F32)
    full = lambda *s: pl.BlockSpec(s, lambda b: (0,) * len(s))
    return pl.pallas_call(
        _p1_body,
        grid=(B,),
        in_specs=[
            pl.BlockSpec((1, CIN, N), lambda b: (b, 0, 0)),
            pl.BlockSpec((1, CIN, M), lambda b: (b, 0, 0)),
            pl.BlockSpec((1, 3, N), lambda b: (b, 0, 0)),
            pl.BlockSpec((1, 3, M), lambda b: (b, 0, 0)),
            full(D, CIN), full(1, D), full(D, CIN), full(1, D),
            full(D, CIN), full(1, D), full(3, 16),
        ],
        out_specs=[
            pl.BlockSpec((1, N, D), lambda b: (b, 0, 0)),
            pl.BlockSpec((1, M, D), lambda b: (b, 0, 0)),
            pl.BlockSpec((1, N, 16), lambda b: (b, 0, 0)),
            pl.BlockSpec((1, M, 16), lambda b: (b, 0, 0)),
        ],
        out_shape=[
            jax.ShapeDtypeStruct((B, N, D), _F32),
            jax.ShapeDtypeStruct((B, M, D), jnp.int32),
            jax.ShapeDtypeStruct((B, N, 16), _F32),
            jax.ShapeDtypeStruct((B, M, 16), _F32),
        ],
    )(fq, fs, pq, ps, qw, qb.reshape(1, D), kw, kb.reshape(1, D),
      vw, vb.reshape(1, D), e3)


TN2 = 256


def _p2_body(pqt, pst, ps, gidx_ref):
    b = pl.program_id(0)
    qt = pqt[0]
    st = pst[0]
    dot = lax.dot_general(qt.astype(jnp.bfloat16), st.astype(jnp.bfloat16),
                          (((1,), (1,)), ((), ())),
                          preferred_element_type=_F32)
    qn = jnp.sum(qt * qt, axis=1, keepdims=True)
    sn = jnp.sum(ps[0] * ps[0], axis=0, keepdims=True)
    negd = 2.0 * dot - qn - sn
    iot = lax.broadcasted_iota(jnp.int32, (TN2, M), 1)
    lane16 = lax.broadcasted_iota(jnp.int32, (TN2, 16), 1)
    acc = jnp.zeros((TN2, 16), jnp.int32)
    for j in range(K):
        sel = jnp.argmax(negd, axis=1).astype(jnp.int32)[:, None]
        acc = jnp.where(lane16 == j, sel, acc)
        negd = jnp.where(iot == sel, -jnp.inf, negd)
    gidx_ref[0] = acc + b * M


def _p2(pqt, pst, ps):
    return pl.pallas_call(
        _p2_body,
        grid=(B, N // TN2),
        in_specs=[
            pl.BlockSpec((1, TN2, 16), lambda b, i: (b, i, 0)),
            pl.BlockSpec((1, M, 16), lambda b, i: (b, 0, 0)),
            pl.BlockSpec((1, 3, M), lambda b, i: (b, 0, 0)),
        ],
        out_specs=pl.BlockSpec((1, TN2, 16), lambda b, i: (b, i, 0)),
        out_shape=jax.ShapeDtypeStruct((B, N, 16), jnp.int32),
    )(pqt, pst, ps)


_NC, _NS = 2, 16
_NW = _NC * _NS
_CH = 256
_PER_W = TOT // _NW
_NCHUNK = _PER_W // _CH


_CHKV = 512
_NCHUNK_KV = _PER_W // _CHKV


def _sc_gather(kvt, pt, gidx):
    mesh = plsc.VectorSubcoreMesh(core_axis_name="c", subcore_axis_name="s")

    @functools.partial(
        pl.kernel, mesh=mesh,
        out_type=[
            jax.ShapeDtypeStruct((TOT, D), jnp.int32),
            jax.ShapeDtypeStruct((TOT, 16), _F32),
        ],
        scratch_types=[
            pltpu.VMEM((_CHKV,), jnp.int32),
            pltpu.VMEM((_CHKV, D), jnp.int32),
            pltpu.VMEM((_CHKV, 16), _F32),
            pltpu.SemaphoreType.DMA,
            pltpu.SemaphoreType.DMA,
        ],
        compiler_params=pltpu.CompilerParams(use_tc_tiling_on_sc=False),
    )
    def run(kv_hbm, pt_hbm, idx_hbm, kvg_hbm, pg_hbm, idx_v, rows, prows,
            sem1, sem2):
        wid = lax.axis_index("s") * _NC + lax.axis_index("c")

        def body(c, _):
            base = pl.multiple_of(wid * _PER_W + c * _CHKV, _CHKV)
            pltpu.sync_copy(idx_hbm.at[pl.ds(base, _CHKV)], idx_v)
            cp1 = pltpu.async_copy(kv_hbm.at[idx_v], rows, sem1)
            cp2 = pltpu.async_copy(pt_hbm.at[idx_v], prows, sem2)
            cp1.wait()
            cp2.wait()
            pltpu.sync_copy(rows, kvg_hbm.at[pl.ds(base, _CHKV)])
            pltpu.sync_copy(prows, pg_hbm.at[pl.ds(base, _CHKV)])
            return _

        lax.fori_loop(0, _NCHUNK_KV, body, None)

    return run(kvt, pt, gidx)


TP = 2048
TQ = TP // K


def _rep_rows(x, reps, width):
    return jnp.broadcast_to(x[:, None, :], (TQ, reps, width)).reshape(
        TQ * reps, width)


def _bn_fold(s1_ref, s2_ref, wt, bias, gamma, beta):
    inv = 1.0 / float(TOT)
    m1 = jnp.sum(s1_ref[...], axis=0, keepdims=True) * inv
    m2 = s2_ref[...] * inv
    t = _dot(m1, wt, ((1,), (0,)))
    q = jnp.sum(wt * _dot(m2, wt, ((1,), (0,))), axis=0, keepdims=True)
    var = q - t * t
    s = gamma * lax.rsqrt(var + 1e-5)
    return wt * s, (bias - (t + bias)) * s + beta


def _p4_body(pg_t, pqt, pw1t, pb1, pgam, pbt,
             pw1f_ref, pb1f_ref, s1_ref, s2_ref):
    i = pl.program_id(0)
    pr = _rep_rows(pqt[...], K, 16) - pg_t[...]
    ps1 = jnp.sum(pr.reshape(TP // 8, 8, 16), axis=0)
    ps2 = _dotbf(pr, pr, ((0,), (0,)))

    @pl.when(i == 0)
    def _():
        s1_ref[...] = jnp.zeros_like(s1_ref)
        s2_ref[...] = jnp.zeros_like(s2_ref)

    s1_ref[...] += ps1
    s2_ref[...] += ps2

    @pl.when(i == pl.num_programs(0) - 1)
    def _():
        wf, bf = _bn_fold(s1_ref, s2_ref, pw1t[...], pb1[...], pgam[...],
                          pbt[...])
        pw1f_ref[...] = wf
        pb1f_ref[...] = bf


def _p4(pgat, pqt_flat, pw1t, pb1, pgam, pbt):
    full = lambda *s: pl.BlockSpec(s, lambda i: (0,) * len(s))
    return pl.pallas_call(
        _p4_body,
        grid=(TOT // TP,),
        in_specs=[
            pl.BlockSpec((TP, 16), lambda i: (i, 0)),
            pl.BlockSpec((TQ, 16), lambda i: (i, 0)),
            full(16, D), full(1, D), full(1, D), full(1, D),
        ],
        out_specs=[
            pl.BlockSpec((16, D), lambda i: (0, 0)),
            pl.BlockSpec((1, D), lambda i: (0, 0)),
        ],
        out_shape=[
            jax.ShapeDtypeStruct((16, D), _F32),
            jax.ShapeDtypeStruct((1, D), _F32),
        ],
        scratch_shapes=[
            pltpu.VMEM((8, 16), _F32),
            pltpu.VMEM((16, 16), _F32),
        ],
    )(pgat, pqt_flat, pw1t, pb1.reshape(1, D), pgam.reshape(1, D),
      pbt.reshape(1, D))


def _pos_emb(pg_t, pqt, pw1f, pb1f, pw2t, pb2):
    pr = _rep_rows(pqt, K, 16) - pg_t
    h = jnp.maximum(_dotbf(pr, pw1f, ((1,), (0,))) + pb1f, 0.0)
    return _dotbf(h, pw2t, ((1,), (0,))) + pb2


def _unpack_k(w):
    return lax.bitcast_convert_type(
        jnp.bitwise_and(w, jnp.int32(-65536)), _F32)


def _unpack_v(w):
    return lax.bitcast_convert_type(lax.shift_left(w, 16), _F32)


def _p5_body(kv_t, pg_t, qt, pqt, pw1f, pb1f, pw2t, pb2,
             aw1t, ab1, agam, abt,
             aw1f_ref, ab1f_ref, sx_ref, sxx_ref):
    i = pl.program_id(0)
    pe = _pos_emb(pg_t[...], pqt[...], pw1f[...], pb1f[...], pw2t[...],
                  pb2[...])
    xv = _rep_rows(qt[...], K, D) - _unpack_k(kv_t[...]) + pe

    @pl.when(i == 0)
    def _():
        sx_ref[...] = jnp.zeros_like(sx_ref)
        sxx_ref[...] = jnp.zeros_like(sxx_ref)

    sx_ref[...] += jnp.sum(xv.reshape(TP // 8, 8, D), axis=0)
    sxx_ref[...] += _dotbf(xv, xv, ((0,), (0,)))

    @pl.when(i == pl.num_programs(0) - 1)
    def _():
        wf, bf = _bn_fold(sx_ref, sxx_ref, aw1t[...], ab1[...], agam[...],
                          abt[...])
        aw1f_ref[...] = wf
        ab1f_ref[...] = bf


def _p5(kvg, pgat, qt_flat, pqt_flat, pw1f, pb1f, pw2t, pb2,
        aw1t, ab1, agam, abt):
    full = lambda *s: pl.BlockSpec(s, lambda i: (0,) * len(s))
    return pl.pallas_call(
        _p5_body,
        grid=(TOT // TP,),
        in_specs=[
            pl.BlockSpec((TP, D), lambda i: (i, 0)),
            pl.BlockSpec((TP, 16), lambda i: (i, 0)),
            pl.BlockSpec((TQ, D), lambda i: (i, 0)),
            pl.BlockSpec((TQ, 16), lambda i: (i, 0)),
            full(16, D), full(1, D), full(D, D), full(1, D),
            full(D, DH), full(1, DH), full(1, DH), full(1, DH),
        ],
        out_specs=[
            pl.BlockSpec((D, DH), lambda i: (0, 0)),
            pl.BlockSpec((1, DH), lambda i: (0, 0)),
        ],
        out_shape=[
            jax.ShapeDtypeStruct((D, DH), _F32),
            jax.ShapeDtypeStruct((1, DH), _F32),
        ],
        scratch_shapes=[
            pltpu.VMEM((8, D), _F32),
            pltpu.VMEM((D, D), _F32),
        ],
    )(kvg, pgat, qt_flat, pqt_flat, pw1f, pb1f, pw2t,
      pb2.reshape(1, D), aw1t, ab1.reshape(1, DH), agam.reshape(1, DH),
      abt.reshape(1, DH))


def _p6_body(kv_t, pg_t, qt, pqt, fq_t, pw1f, pb1f, pw2t, pb2,
             aw1f, ab1f, aw2t, ab2, ewt, ebc, out_ref):
    pe = _pos_emb(pg_t[...], pqt[...], pw1f[...], pb1f[...], pw2t[...],
                  pb2[...])
    w_kv = kv_t[...]
    xv = _rep_rows(qt[...], K, D) - _unpack_k(w_kv) + pe
    vgp = _unpack_v(w_kv) + pe
    a = jnp.maximum(_dotbf(xv, aw1f[...], ((1,), (0,))) + ab1f[...], 0.0)
    lg = _dotbf(a, aw2t[...], ((1,), (0,))) + ab2[...]
    e = jnp.exp(lg)
    den = jnp.sum(e.reshape(TQ, K, D), axis=1)
    inv = 1.0 / den
    w = e * _rep_rows(inv, K, D) * vgp
    agg = jnp.sum(w.reshape(TQ, K, D), axis=1)
    out_ref[0] = _dotbf(ewt[...], agg, ((1,), (1,))) + ebc[...] + fq_t[0]


def _p6(kvg, pgat, qt_flat, pqt_flat, fq, pw1f, pb1f, pw2t, pb2,
        aw1f, ab1f, aw2t, ab2, ew, eb):
    full = lambda *s: pl.BlockSpec(s, lambda b, i: (0,) * len(s))
    nt = N // TQ
    return pl.pallas_call(
        _p6_body,
        grid=(B, nt),
        in_specs=[
            pl.BlockSpec((TP, D), lambda b, i: (b * nt + i, 0)),
            pl.BlockSpec((TP, 16), lambda b, i: (b * nt + i, 0)),
            pl.BlockSpec((TQ, D), lambda b, i: (b * nt + i, 0)),
            pl.BlockSpec((TQ, 16), lambda b, i: (b * nt + i, 0)),
            pl.BlockSpec((1, CIN, TQ), lambda b, i: (b, 0, i)),
            full(16, D), full(1, D), full(D, D), full(1, D),
            full(D, DH), full(1, DH), full(DH, D), full(1, D),
            full(CIN, D), full(CIN, 1),
        ],
        out_specs=pl.BlockSpec((1, CIN, TQ), lambda b, i: (b, 0, i)),
        out_shape=jax.ShapeDtypeStruct((B, CIN, N), _F32),
    )(kvg, pgat, qt_flat, pqt_flat, fq, pw1f, pb1f, pw2t,
      pb2.reshape(1, D), aw1f, ab1f.reshape(1, DH), aw2t,
      ab2.reshape(1, D), ew, eb.reshape(CIN, 1))


def kernel(pq, fq, ps, fs, qw, qb, kw, kb, vw, vb, pw1, pb1, pg, pbt,
           pw2, pb2, aw1, ab1, ag, abt, aw2, ab2, ew, eb):
    qt, kvt, pqt, pst = _p1(fq, fs, pq, ps, qw, qb, kw, kb, vw, vb)
    gidx = _p2(pqt, pst, ps)
    gfl = gidx.reshape(TOT)
    kvg, pgat = _sc_gather(kvt.reshape(B * M, D),
                           pst.reshape(B * M, 16), gfl)
    pw1t = jnp.concatenate([pw1.T, jnp.zeros((16 - 3, D), _F32)], axis=0)
    pw1f, pb1f = _p4(pgat, pqt.reshape(B * N, 16), pw1t, pb1, pg, pbt)
    qtf = qt.reshape(B * N, D)
    pqf = pqt.reshape(B * N, 16)
    aw1f, ab1f = _p5(kvg, pgat, qtf, pqf, pw1f, pb1f, pw2.T, pb2,
                     aw1.T, ab1, ag, abt)
    return _p6(kvg, pgat, qtf, pqf, fq, pw1f, pb1f, pw2.T, pb2,
               aw1f, ab1f, aw2.T, ab2, ew, eb)


# speedup vs baseline: 1.0040x; 1.0040x over previous
import functools

import jax
import jax.numpy as jnp
from jax import lax
from jax.experimental import pallas as pl
from jax.experimental.pallas import tpu as pltpu
from jax.experimental.pallas import tpu_sc as plsc

B = 4
N = 2048
M = 2048
CIN = 256
D = 128
K = 16
DH = 512
TOT = B * N * K

_HI = lax.Precision.HIGHEST
_F32 = jnp.float32


def _dot(a, b, dims):
    return lax.dot_general(a, b, (dims, ((), ())), precision=_HI,
                           preferred_element_type=_F32)


def _dotbf(a, b, dims):
    return lax.dot_general(a.astype(jnp.bfloat16), b.astype(jnp.bfloat16),
                           (dims, ((), ())), preferred_element_type=_F32)


def _bf_bits(x):
    return lax.bitcast_convert_type(
        x.astype(jnp.bfloat16).astype(_F32), jnp.int32)


def _p1_body(fq, fs, pq, ps, qw, qb, kw, kb, vw, vb, e3,
             qt_ref, kvt_ref, pqt_ref, pst_ref):
    f_q = fq[0]
    f_s = fs[0]
    qt_ref[0] = _dotbf(f_q, qw[...], ((0,), (1,))) + qb[...]
    kres = _dotbf(f_s, kw[...], ((0,), (1,))) + kb[...]
    vres = _dotbf(f_s, vw[...], ((0,), (1,))) + vb[...]
    kvt_ref[0] = jnp.bitwise_or(
        _bf_bits(kres), lax.shift_right_logical(_bf_bits(vres), 16))
    pqt_ref[0] = _dot(pq[0], e3[...], ((0,), (0,)))
    pst_ref[0] = _dot(ps[0], e3[...], ((0,), (0,)))


def _p1(fq, fs, pq, ps, qw, qb, kw, kb, vw, vb):
    e3 = jnp.eye(3, 16, dtype=_F32)
    full = lambda *s: pl.BlockSpec(s, lambda b: (0,) * len(s))
    return pl.pallas_call(
        _p1_body,
        grid=(B,),
        in_specs=[
            pl.BlockSpec((1, CIN, N), lambda b: (b, 0, 0)),
            pl.BlockSpec((1, CIN, M), lambda b: (b, 0, 0)),
            pl.BlockSpec((1, 3, N), lambda b: (b, 0, 0)),
            pl.BlockSpec((1, 3, M), lambda b: (b, 0, 0)),
            full(D, CIN), full(1, D), full(D, CIN), full(1, D),
            full(D, CIN), full(1, D), full(3, 16),
        ],
        out_specs=[
            pl.BlockSpec((1, N, D), lambda b: (b, 0, 0)),
            pl.BlockSpec((1, M, D), lambda b: (b, 0, 0)),
            pl.BlockSpec((1, N, 16), lambda b: (b, 0, 0)),
            pl.BlockSpec((1, M, 16), lambda b: (b, 0, 0)),
        ],
        out_shape=[
            jax.ShapeDtypeStruct((B, N, D), _F32),
            jax.ShapeDtypeStruct((B, M, D), jnp.int32),
            jax.ShapeDtypeStruct((B, N, 16), _F32),
            jax.ShapeDtypeStruct((B, M, 16), _F32),
        ],
    )(fq, fs, pq, ps, qw, qb.reshape(1, D), kw, kb.reshape(1, D),
      vw, vb.reshape(1, D), e3)


TN2 = 256


def _p2_body(pqt, pst, ps, gidx_ref):
    b = pl.program_id(0)
    qt = pqt[0]
    st = pst[0]
    dot = lax.dot_general(qt.astype(jnp.bfloat16), st.astype(jnp.bfloat16),
                          (((1,), (1,)), ((), ())),
                          preferred_element_type=_F32)
    qn = jnp.sum(qt * qt, axis=1, keepdims=True)
    sn = jnp.sum(ps[0] * ps[0], axis=0, keepdims=True)
    negd = 2.0 * dot - qn - sn
    iot = lax.broadcasted_iota(jnp.int32, (TN2, M), 1)
    lane16 = lax.broadcasted_iota(jnp.int32, (TN2, 16), 1)
    acc = jnp.zeros((TN2, 16), jnp.int32)
    for j in range(K):
        sel = jnp.argmax(negd, axis=1).astype(jnp.int32)[:, None]
        acc = jnp.where(lane16 == j, sel, acc)
        negd = jnp.where(iot == sel, -jnp.inf, negd)
    gidx_ref[0] = acc + b * M


def _p2(pqt, pst, ps):
    return pl.pallas_call(
        _p2_body,
        grid=(B, N // TN2),
        in_specs=[
            pl.BlockSpec((1, TN2, 16), lambda b, i: (b, i, 0)),
            pl.BlockSpec((1, M, 16), lambda b, i: (b, 0, 0)),
            pl.BlockSpec((1, 3, M), lambda b, i: (b, 0, 0)),
        ],
        out_specs=pl.BlockSpec((1, TN2, 16), lambda b, i: (b, i, 0)),
        out_shape=jax.ShapeDtypeStruct((B, N, 16), jnp.int32),
    )(pqt, pst, ps)


_NC, _NS = 2, 16
_NW = _NC * _NS
_CH = 256
_PER_W = TOT // _NW
_NCHUNK = _PER_W // _CH


_CHKV = 512
_NCHUNK_KV = _PER_W // _CHKV


def _sc_gather(kvt, pt, gidx):
    mesh = plsc.VectorSubcoreMesh(core_axis_name="c", subcore_axis_name="s")

    @functools.partial(
        pl.kernel, mesh=mesh,
        out_type=[
            jax.ShapeDtypeStruct((TOT, D), jnp.int32),
            jax.ShapeDtypeStruct((TOT, 16), _F32),
        ],
        scratch_types=[
            pltpu.VMEM((_CHKV,), jnp.int32),
            pltpu.VMEM((_CHKV, D), jnp.int32),
            pltpu.VMEM((_CHKV, 16), _F32),
            pltpu.SemaphoreType.DMA,
            pltpu.SemaphoreType.DMA,
        ],
        compiler_params=pltpu.CompilerParams(use_tc_tiling_on_sc=False),
    )
    def run(kv_hbm, pt_hbm, idx_hbm, kvg_hbm, pg_hbm, idx_v, rows, prows,
            sem1, sem2):
        wid = lax.axis_index("s") * _NC + lax.axis_index("c")

        def body(c, _):
            base = pl.multiple_of(wid * _PER_W + c * _CHKV, _CHKV)
            pltpu.sync_copy(idx_hbm.at[pl.ds(base, _CHKV)], idx_v)
            cp1 = pltpu.async_copy(kv_hbm.at[idx_v], rows, sem1)
            cp2 = pltpu.async_copy(pt_hbm.at[idx_v], prows, sem2)
            cp1.wait()
            cp2.wait()
            pltpu.sync_copy(rows, kvg_hbm.at[pl.ds(base, _CHKV)])
            pltpu.sync_copy(prows, pg_hbm.at[pl.ds(base, _CHKV)])
            return _

        lax.fori_loop(0, _NCHUNK_KV, body, None)

    return run(kvt, pt, gidx)


TP = 2048
TQ = TP // K


def _rep_rows(x, reps, width):
    return jnp.broadcast_to(x[:, None, :], (TQ, reps, width)).reshape(
        TQ * reps, width)


def _bn_fold(s1_ref, s2_ref, wt, bias, gamma, beta):
    inv = 1.0 / float(TOT)
    m1 = jnp.sum(s1_ref[...], axis=0, keepdims=True) * inv
    m2 = s2_ref[...] * inv
    t = _dot(m1, wt, ((1,), (0,)))
    q = jnp.sum(wt * _dot(m2, wt, ((1,), (0,))), axis=0, keepdims=True)
    var = q - t * t
    s = gamma * lax.rsqrt(var + 1e-5)
    return wt * s, (bias - (t + bias)) * s + beta


def _pos_emb(pg_t, pqt, pw1f, pb1f, pw2t, pb2):
    pr = _rep_rows(pqt, K, 16) - pg_t
    h = jnp.maximum(_dotbf(pr, pw1f, ((1,), (0,))) + pb1f, 0.0)
    return _dotbf(h, pw2t, ((1,), (0,))) + pb2


def _unpack_k(w):
    return lax.bitcast_convert_type(
        jnp.bitwise_and(w, jnp.int32(-65536)), _F32)


def _unpack_v(w):
    return lax.bitcast_convert_type(lax.shift_left(w, 16), _F32)


def _p45_body(kv_t, pg_t, qt, pqt, pw1t, pb1, pgam, pbt, pw2t, pb2,
              aw1t, ab1, agam, abt,
              aw1f_ref, ab1f_ref, pw1f_ref, pb1f_ref,
              s1_ref, s2_ref, sx_ref, sxx_ref):
    t = pl.program_id(0)
    i = pl.program_id(1)
    last = pl.num_programs(1) - 1

    @pl.when(t == 0)
    def _():
        pr = _rep_rows(pqt[...], K, 16) - pg_t[...]
        ps1 = jnp.sum(pr.reshape(TP // 8, 8, 16), axis=0)
        ps2 = _dotbf(pr, pr, ((0,), (0,)))

        @pl.when(i == 0)
        def _():
            s1_ref[...] = jnp.zeros_like(s1_ref)
            s2_ref[...] = jnp.zeros_like(s2_ref)

        s1_ref[...] += ps1
        s2_ref[...] += ps2

        @pl.when(i == last)
        def _():
            wf, bf = _bn_fold(s1_ref, s2_ref, pw1t[...], pb1[...],
                              pgam[...], pbt[...])
            pw1f_ref[...] = wf
            pb1f_ref[...] = bf

    @pl.when(t == 1)
    def _():
        pe = _pos_emb(pg_t[...], pqt[...], pw1f_ref[...], pb1f_ref[...],
                      pw2t[...], pb2[...])
        xv = _rep_rows(qt[...], K, D) - _unpack_k(kv_t[...]) + pe

        @pl.when(i == 0)
        def _():
            sx_ref[...] = jnp.zeros_like(sx_ref)
            sxx_ref[...] = jnp.zeros_like(sxx_ref)

        sx_ref[...] += jnp.sum(xv.reshape(TP // 8, 8, D), axis=0)
        sxx_ref[...] += _dotbf(xv, xv, ((0,), (0,)))

        @pl.when(i == last)
        def _():
            wf, bf = _bn_fold(sx_ref, sxx_ref, aw1t[...], ab1[...],
                              agam[...], abt[...])
            aw1f_ref[...] = wf
            ab1f_ref[...] = bf


def _p45(kvg, pgat, qt_flat, pqt_flat, pw1t, pb1, pgam, pbt, pw2t, pb2,
         aw1t, ab1, agam, abt):
    full = lambda *s: pl.BlockSpec(s, lambda t, i: (0,) * len(s))
    return pl.pallas_call(
        _p45_body,
        grid=(2, TOT // TP),
        in_specs=[
            pl.BlockSpec((TP, D), lambda t, i: (i * t, 0)),
            pl.BlockSpec((TP, 16), lambda t, i: (i, 0)),
            pl.BlockSpec((TQ, D), lambda t, i: (i * t, 0)),
            pl.BlockSpec((TQ, 16), lambda t, i: (i, 0)),
            full(16, D), full(1, D), full(1, D), full(1, D),
            full(D, D), full(1, D),
            full(D, DH), full(1, DH), full(1, DH), full(1, DH),
        ],
        out_specs=[
            pl.BlockSpec((D, DH), lambda t, i: (0, 0)),
            pl.BlockSpec((1, DH), lambda t, i: (0, 0)),
            pl.BlockSpec((16, D), lambda t, i: (0, 0)),
            pl.BlockSpec((1, D), lambda t, i: (0, 0)),
        ],
        out_shape=[
            jax.ShapeDtypeStruct((D, DH), _F32),
            jax.ShapeDtypeStruct((1, DH), _F32),
            jax.ShapeDtypeStruct((16, D), _F32),
            jax.ShapeDtypeStruct((1, D), _F32),
        ],
        scratch_shapes=[
            pltpu.VMEM((8, 16), _F32),
            pltpu.VMEM((16, 16), _F32),
            pltpu.VMEM((8, D), _F32),
            pltpu.VMEM((D, D), _F32),
        ],
    )(kvg, pgat, qt_flat, pqt_flat, pw1t, pb1.reshape(1, D),
      pgam.reshape(1, D), pbt.reshape(1, D), pw2t, pb2.reshape(1, D),
      aw1t, ab1.reshape(1, DH), agam.reshape(1, DH), abt.reshape(1, DH))


def _p6_body(kv_t, pg_t, qt, pqt, fq_t, pw1f, pb1f, pw2t, pb2,
             aw1f, ab1f, aw2t, ab2, ewt, ebc, out_ref):
    pe = _pos_emb(pg_t[...], pqt[...], pw1f[...], pb1f[...], pw2t[...],
                  pb2[...])
    w_kv = kv_t[...]
    xv = _rep_rows(qt[...], K, D) - _unpack_k(w_kv) + pe
    vgp = _unpack_v(w_kv) + pe
    a = jnp.maximum(_dotbf(xv, aw1f[...], ((1,), (0,))) + ab1f[...], 0.0)
    lg = _dotbf(a, aw2t[...], ((1,), (0,))) + ab2[...]
    e = jnp.exp(lg)
    den = jnp.sum(e.reshape(TQ, K, D), axis=1)
    inv = 1.0 / den
    w = e * _rep_rows(inv, K, D) * vgp
    agg = jnp.sum(w.reshape(TQ, K, D), axis=1)
    out_ref[0] = _dotbf(ewt[...], agg, ((1,), (1,))) + ebc[...] + fq_t[0]


def _p6(kvg, pgat, qt_flat, pqt_flat, fq, pw1f, pb1f, pw2t, pb2,
        aw1f, ab1f, aw2t, ab2, ew, eb):
    full = lambda *s: pl.BlockSpec(s, lambda b, i: (0,) * len(s))
    nt = N // TQ
    return pl.pallas_call(
        _p6_body,
        grid=(B, nt),
        in_specs=[
            pl.BlockSpec((TP, D), lambda b, i: (b * nt + i, 0)),
            pl.BlockSpec((TP, 16), lambda b, i: (b * nt + i, 0)),
            pl.BlockSpec((TQ, D), lambda b, i: (b * nt + i, 0)),
            pl.BlockSpec((TQ, 16), lambda b, i: (b * nt + i, 0)),
            pl.BlockSpec((1, CIN, TQ), lambda b, i: (b, 0, i)),
            full(16, D), full(1, D), full(D, D), full(1, D),
            full(D, DH), full(1, DH), full(DH, D), full(1, D),
            full(CIN, D), full(CIN, 1),
        ],
        out_specs=pl.BlockSpec((1, CIN, TQ), lambda b, i: (b, 0, i)),
        out_shape=jax.ShapeDtypeStruct((B, CIN, N), _F32),
    )(kvg, pgat, qt_flat, pqt_flat, fq, pw1f, pb1f, pw2t,
      pb2.reshape(1, D), aw1f, ab1f.reshape(1, DH), aw2t,
      ab2.reshape(1, D), ew, eb.reshape(CIN, 1))


def kernel(pq, fq, ps, fs, qw, qb, kw, kb, vw, vb, pw1, pb1, pg, pbt,
           pw2, pb2, aw1, ab1, ag, abt, aw2, ab2, ew, eb):
    qt, kvt, pqt, pst = _p1(fq, fs, pq, ps, qw, qb, kw, kb, vw, vb)
    gidx = _p2(pqt, pst, ps)
    gfl = gidx.reshape(TOT)
    kvg, pgat = _sc_gather(kvt.reshape(B * M, D),
                           pst.reshape(B * M, 16), gfl)
    pw1t = jnp.concatenate([pw1.T, jnp.zeros((16 - 3, D), _F32)], axis=0)
    qtf = qt.reshape(B * N, D)
    pqf = pqt.reshape(B * N, 16)
    aw1f, ab1f, pw1f, pb1f = _p45(kvg, pgat, qtf, pqf, pw1t, pb1, pg, pbt,
                                  pw2.T, pb2, aw1.T, ab1, ag, abt)
    return _p6(kvg, pgat, qtf, pqf, fq, pw1f, pb1f, pw2.T, pb2,
               aw1f, ab1f, aw2.T, ab2, ew, eb)


# speedup vs baseline: 1.0092x; 1.0052x over previous
import functools

import jax
import jax.numpy as jnp
from jax import lax
from jax.experimental import pallas as pl
from jax.experimental.pallas import tpu as pltpu
from jax.experimental.pallas import tpu_sc as plsc

B = 4
N = 2048
M = 2048
CIN = 256
D = 128
K = 16
DH = 512
TOT = B * N * K

_HI = lax.Precision.HIGHEST
_F32 = jnp.float32


def _dot(a, b, dims):
    return lax.dot_general(a, b, (dims, ((), ())), precision=_HI,
                           preferred_element_type=_F32)


def _dotbf(a, b, dims):
    return lax.dot_general(a.astype(jnp.bfloat16), b.astype(jnp.bfloat16),
                           (dims, ((), ())), preferred_element_type=_F32)


def _bf_bits(x):
    return lax.bitcast_convert_type(
        x.astype(jnp.bfloat16).astype(_F32), jnp.int32)


def _p1_body(fq, fs, pq, ps, qw, qb, kw, kb, vw, vb, e3,
             qt_ref, kvt_ref, pqt_ref, pst_ref):
    f_q = fq[0]
    f_s = fs[0]
    qt_ref[0] = _dotbf(f_q, qw[...], ((0,), (1,))) + qb[...]
    kres = _dotbf(f_s, kw[...], ((0,), (1,))) + kb[...]
    vres = _dotbf(f_s, vw[...], ((0,), (1,))) + vb[...]
    kvt_ref[0] = jnp.bitwise_or(
        _bf_bits(kres), lax.shift_right_logical(_bf_bits(vres), 16))
    pqt_ref[0] = _dot(pq[0], e3[...], ((0,), (0,)))
    pst_ref[0] = _dot(ps[0], e3[...], ((0,), (0,)))


def _p1(fq, fs, pq, ps, qw, qb, kw, kb, vw, vb):
    e3 = jnp.eye(3, 16, dtype=_F32)
    full = lambda *s: pl.BlockSpec(s, lambda b: (0,) * len(s))
    return pl.pallas_call(
        _p1_body,
        grid=(B,),
        in_specs=[
            pl.BlockSpec((1, CIN, N), lambda b: (b, 0, 0)),
            pl.BlockSpec((1, CIN, M), lambda b: (b, 0, 0)),
            pl.BlockSpec((1, 3, N), lambda b: (b, 0, 0)),
            pl.BlockSpec((1, 3, M), lambda b: (b, 0, 0)),
            full(D, CIN), full(1, D), full(D, CIN), full(1, D),
            full(D, CIN), full(1, D), full(3, 16),
        ],
        out_specs=[
            pl.BlockSpec((1, N, D), lambda b: (b, 0, 0)),
            pl.BlockSpec((1, M, D), lambda b: (b, 0, 0)),
            pl.BlockSpec((1, N, 16), lambda b: (b, 0, 0)),
            pl.BlockSpec((1, M, 16), lambda b: (b, 0, 0)),
        ],
        out_shape=[
            jax.ShapeDtypeStruct((B, N, D), _F32),
            jax.ShapeDtypeStruct((B, M, D), jnp.int32),
            jax.ShapeDtypeStruct((B, N, 16), _F32),
            jax.ShapeDtypeStruct((B, M, 16), _F32),
        ],
    )(fq, fs, pq, ps, qw, qb.reshape(1, D), kw, kb.reshape(1, D),
      vw, vb.reshape(1, D), e3)


TN2 = 512


def _p2_body(pqt, pst, ps, gidx_ref):
    b = pl.program_id(0)
    qt = pqt[0]
    st = pst[0]
    dot = lax.dot_general(qt.astype(jnp.bfloat16), st.astype(jnp.bfloat16),
                          (((1,), (1,)), ((), ())),
                          preferred_element_type=_F32)
    qn = jnp.sum(qt * qt, axis=1, keepdims=True)
    sn = jnp.sum(ps[0] * ps[0], axis=0, keepdims=True)
    negd = 2.0 * dot - qn - sn
    iot = lax.broadcasted_iota(jnp.int32, (TN2, M), 1)
    lane16 = lax.broadcasted_iota(jnp.int32, (TN2, 16), 1)
    acc = jnp.zeros((TN2, 16), jnp.int32)
    for j in range(K):
        sel = jnp.argmax(negd, axis=1).astype(jnp.int32)[:, None]
        acc = jnp.where(lane16 == j, sel, acc)
        negd = jnp.where(iot == sel, -jnp.inf, negd)
    gidx_ref[0] = acc + b * M


def _p2(pqt, pst, ps):
    return pl.pallas_call(
        _p2_body,
        grid=(B, N // TN2),
        in_specs=[
            pl.BlockSpec((1, TN2, 16), lambda b, i: (b, i, 0)),
            pl.BlockSpec((1, M, 16), lambda b, i: (b, 0, 0)),
            pl.BlockSpec((1, 3, M), lambda b, i: (b, 0, 0)),
        ],
        out_specs=pl.BlockSpec((1, TN2, 16), lambda b, i: (b, i, 0)),
        out_shape=jax.ShapeDtypeStruct((B, N, 16), jnp.int32),
    )(pqt, pst, ps)


_NC, _NS = 2, 16
_NW = _NC * _NS
_CH = 256
_PER_W = TOT // _NW
_NCHUNK = _PER_W // _CH


_CHKV = 512
_NCHUNK_KV = _PER_W // _CHKV


def _sc_gather(kvt, pt, gidx):
    mesh = plsc.VectorSubcoreMesh(core_axis_name="c", subcore_axis_name="s")

    @functools.partial(
        pl.kernel, mesh=mesh,
        out_type=[
            jax.ShapeDtypeStruct((TOT, D), jnp.int32),
            jax.ShapeDtypeStruct((TOT, 16), _F32),
        ],
        scratch_types=[
            pltpu.VMEM((_CHKV,), jnp.int32),
            pltpu.VMEM((_CHKV, D), jnp.int32),
            pltpu.VMEM((_CHKV, 16), _F32),
            pltpu.SemaphoreType.DMA,
            pltpu.SemaphoreType.DMA,
        ],
        compiler_params=pltpu.CompilerParams(use_tc_tiling_on_sc=False),
    )
    def run(kv_hbm, pt_hbm, idx_hbm, kvg_hbm, pg_hbm, idx_v, rows, prows,
            sem1, sem2):
        wid = lax.axis_index("s") * _NC + lax.axis_index("c")

        def body(c, _):
            base = pl.multiple_of(wid * _PER_W + c * _CHKV, _CHKV)
            pltpu.sync_copy(idx_hbm.at[pl.ds(base, _CHKV)], idx_v)
            cp1 = pltpu.async_copy(kv_hbm.at[idx_v], rows, sem1)
            cp2 = pltpu.async_copy(pt_hbm.at[idx_v], prows, sem2)
            cp1.wait()
            cp2.wait()
            pltpu.sync_copy(rows, kvg_hbm.at[pl.ds(base, _CHKV)])
            pltpu.sync_copy(prows, pg_hbm.at[pl.ds(base, _CHKV)])
            return _

        lax.fori_loop(0, _NCHUNK_KV, body, None)

    return run(kvt, pt, gidx)


TP = 2048
TQ = TP // K


def _rep_rows(x, reps, width):
    return jnp.broadcast_to(x[:, None, :], (TQ, reps, width)).reshape(
        TQ * reps, width)


def _bn_fold(s1_ref, s2_ref, wt, bias, gamma, beta):
    inv = 1.0 / float(TOT)
    m1 = jnp.sum(s1_ref[...], axis=0, keepdims=True) * inv
    m2 = s2_ref[...] * inv
    t = _dot(m1, wt, ((1,), (0,)))
    q = jnp.sum(wt * _dot(m2, wt, ((1,), (0,))), axis=0, keepdims=True)
    var = q - t * t
    s = gamma * lax.rsqrt(var + 1e-5)
    return wt * s, (bias - (t + bias)) * s + beta


def _pos_emb(pg_t, pqt, pw1f, pb1f, pw2t, pb2):
    pr = _rep_rows(pqt, K, 16) - pg_t
    h = jnp.maximum(_dotbf(pr, pw1f, ((1,), (0,))) + pb1f, 0.0)
    return _dotbf(h, pw2t, ((1,), (0,))) + pb2


def _unpack_k(w):
    return lax.bitcast_convert_type(
        jnp.bitwise_and(w, jnp.int32(-65536)), _F32)


def _unpack_v(w):
    return lax.bitcast_convert_type(lax.shift_left(w, 16), _F32)


def _p45_body(kv_t, pg_t, qt, pqt, pw1t, pb1, pgam, pbt, pw2t, pb2,
              aw1t, ab1, agam, abt,
              aw1f_ref, ab1f_ref, pw1f_ref, pb1f_ref,
              s1_ref, s2_ref, sx_ref, sxx_ref):
    t = pl.program_id(0)
    i = pl.program_id(1)
    last = pl.num_programs(1) - 1

    @pl.when(t == 0)
    def _():
        pr = _rep_rows(pqt[...], K, 16) - pg_t[...]
        ps1 = jnp.sum(pr.reshape(TP // 8, 8, 16), axis=0)
        ps2 = _dotbf(pr, pr, ((0,), (0,)))

        @pl.when(i == 0)
        def _():
            s1_ref[...] = jnp.zeros_like(s1_ref)
            s2_ref[...] = jnp.zeros_like(s2_ref)

        s1_ref[...] += ps1
        s2_ref[...] += ps2

        @pl.when(i == last)
        def _():
            wf, bf = _bn_fold(s1_ref, s2_ref, pw1t[...], pb1[...],
                              pgam[...], pbt[...])
            pw1f_ref[...] = wf
            pb1f_ref[...] = bf

    @pl.when(t == 1)
    def _():
        pe = _pos_emb(pg_t[...], pqt[...], pw1f_ref[...], pb1f_ref[...],
                      pw2t[...], pb2[...])
        xv = _rep_rows(qt[...], K, D) - _unpack_k(kv_t[...]) + pe

        @pl.when(i == 0)
        def _():
            sx_ref[...] = jnp.zeros_like(sx_ref)
            sxx_ref[...] = jnp.zeros_like(sxx_ref)

        sx_ref[...] += jnp.sum(xv.reshape(TP // 8, 8, D), axis=0)
        sxx_ref[...] += _dotbf(xv, xv, ((0,), (0,)))

        @pl.when(i == last)
        def _():
            wf, bf = _bn_fold(sx_ref, sxx_ref, aw1t[...], ab1[...],
                              agam[...], abt[...])
            aw1f_ref[...] = wf
            ab1f_ref[...] = bf


def _p45(kvg, pgat, qt_flat, pqt_flat, pw1t, pb1, pgam, pbt, pw2t, pb2,
         aw1t, ab1, agam, abt):
    full = lambda *s: pl.BlockSpec(s, lambda t, i: (0,) * len(s))
    return pl.pallas_call(
        _p45_body,
        grid=(2, TOT // TP),
        in_specs=[
            pl.BlockSpec((TP, D), lambda t, i: (i * t, 0)),
            pl.BlockSpec((TP, 16), lambda t, i: (i, 0)),
            pl.BlockSpec((TQ, D), lambda t, i: (i * t, 0)),
            pl.BlockSpec((TQ, 16), lambda t, i: (i, 0)),
            full(16, D), full(1, D), full(1, D), full(1, D),
            full(D, D), full(1, D),
            full(D, DH), full(1, DH), full(1, DH), full(1, DH),
        ],
        out_specs=[
            pl.BlockSpec((D, DH), lambda t, i: (0, 0)),
            pl.BlockSpec((1, DH), lambda t, i: (0, 0)),
            pl.BlockSpec((16, D), lambda t, i: (0, 0)),
            pl.BlockSpec((1, D), lambda t, i: (0, 0)),
        ],
        out_shape=[
            jax.ShapeDtypeStruct((D, DH), _F32),
            jax.ShapeDtypeStruct((1, DH), _F32),
            jax.ShapeDtypeStruct((16, D), _F32),
            jax.ShapeDtypeStruct((1, D), _F32),
        ],
        scratch_shapes=[
            pltpu.VMEM((8, 16), _F32),
            pltpu.VMEM((16, 16), _F32),
            pltpu.VMEM((8, D), _F32),
            pltpu.VMEM((D, D), _F32),
        ],
    )(kvg, pgat, qt_flat, pqt_flat, pw1t, pb1.reshape(1, D),
      pgam.reshape(1, D), pbt.reshape(1, D), pw2t, pb2.reshape(1, D),
      aw1t, ab1.reshape(1, DH), agam.reshape(1, DH), abt.reshape(1, DH))


def _p6_body(kv_t, pg_t, qt, pqt, fq_t, pw1f, pb1f, pw2t, pb2,
             aw1f, ab1f, aw2t, ab2, ewt, ebc, out_ref):
    pe = _pos_emb(pg_t[...], pqt[...], pw1f[...], pb1f[...], pw2t[...],
                  pb2[...])
    w_kv = kv_t[...]
    xv = _rep_rows(qt[...], K, D) - _unpack_k(w_kv) + pe
    vgp = _unpack_v(w_kv) + pe
    a = jnp.maximum(_dotbf(xv, aw1f[...], ((1,), (0,))) + ab1f[...], 0.0)
    lg = _dotbf(a, aw2t[...], ((1,), (0,))) + ab2[...]
    e = jnp.exp(lg)
    den = jnp.sum(e.reshape(TQ, K, D), axis=1)
    inv = 1.0 / den
    w = e * _rep_rows(inv, K, D) * vgp
    agg = jnp.sum(w.reshape(TQ, K, D), axis=1)
    out_ref[0] = _dotbf(ewt[...], agg, ((1,), (1,))) + ebc[...] + fq_t[0]


def _p6(kvg, pgat, qt_flat, pqt_flat, fq, pw1f, pb1f, pw2t, pb2,
        aw1f, ab1f, aw2t, ab2, ew, eb):
    full = lambda *s: pl.BlockSpec(s, lambda b, i: (0,) * len(s))
    nt = N // TQ
    return pl.pallas_call(
        _p6_body,
        grid=(B, nt),
        in_specs=[
            pl.BlockSpec((TP, D), lambda b, i: (b * nt + i, 0)),
            pl.BlockSpec((TP, 16), lambda b, i: (b * nt + i, 0)),
            pl.BlockSpec((TQ, D), lambda b, i: (b * nt + i, 0)),
            pl.BlockSpec((TQ, 16), lambda b, i: (b * nt + i, 0)),
            pl.BlockSpec((1, CIN, TQ), lambda b, i: (b, 0, i)),
            full(16, D), full(1, D), full(D, D), full(1, D),
            full(D, DH), full(1, DH), full(DH, D), full(1, D),
            full(CIN, D), full(CIN, 1),
        ],
        out_specs=pl.BlockSpec((1, CIN, TQ), lambda b, i: (b, 0, i)),
        out_shape=jax.ShapeDtypeStruct((B, CIN, N), _F32),
    )(kvg, pgat, qt_flat, pqt_flat, fq, pw1f, pb1f, pw2t,
      pb2.reshape(1, D), aw1f, ab1f.reshape(1, DH), aw2t,
      ab2.reshape(1, D), ew, eb.reshape(CIN, 1))


def kernel(pq, fq, ps, fs, qw, qb, kw, kb, vw, vb, pw1, pb1, pg, pbt,
           pw2, pb2, aw1, ab1, ag, abt, aw2, ab2, ew, eb):
    qt, kvt, pqt, pst = _p1(fq, fs, pq, ps, qw, qb, kw, kb, vw, vb)
    gidx = _p2(pqt, pst, ps)
    gfl = gidx.reshape(TOT)
    kvg, pgat = _sc_gather(kvt.reshape(B * M, D),
                           pst.reshape(B * M, 16), gfl)
    pw1t = jnp.concatenate([pw1.T, jnp.zeros((16 - 3, D), _F32)], axis=0)
    qtf = qt.reshape(B * N, D)
    pqf = pqt.reshape(B * N, 16)
    aw1f, ab1f, pw1f, pb1f = _p45(kvg, pgat, qtf, pqf, pw1t, pb1, pg, pbt,
                                  pw2.T, pb2, aw1.T, ab1, ag, abt)
    return _p6(kvg, pgat, qtf, pqf, fq, pw1f, pb1f, pw2.T, pb2,
               aw1f, ab1f, aw2.T, ab2, ew, eb)
